# baseline (speedup 1.0000x reference)
import jax
import jax.numpy as jnp
from jax import lax
from jax.experimental import pallas as pl
from jax.experimental.pallas import tpu as pltpu
from jax.experimental.pallas import tpu_sc as plsc

_EMB = 64
_B = 1024
_L = 200

_NC = 2
_NS = 16
_NW = _NC * _NS
_BPW = _B // _NW
_CHA = 128
_CHB = _L - _CHA

_TPW = _BPW * _L


def _posseg_body(pos_ref, seg_ref, out_ref):
    seg = seg_ref[...]
    pos = pos_ref[...]
    out_ref[...] = seg[:, None, :] + pos[None, :, :]


def _posseg(pos_table, seg_table):
    out = pl.pallas_call(
        _posseg_body,
        out_shape=jax.ShapeDtypeStruct((3, _L, _EMB), jnp.float32),
    )(pos_table, seg_table)
    return out.reshape(3 * _L, _EMB)


def _sc_body(x_hbm, s_hbm, tt_hbm, ps_hbm, out_hbm,
             idx_v, psidx_v, bufa_v, bufb_v, toka_sems, tokb_sems,
             adda_sems, addb_sems, wra_sems, wrb_sems, seg_sem):
    w = lax.axis_index("s") * _NC + lax.axis_index("c")
    b0 = w * _BPW

    pltpu.sync_copy(x_hbm.at[pl.ds(b0, _BPW)], idx_v)
    for r in range(_BPW):
        pltpu.async_copy(s_hbm.at[b0 + r], psidx_v.at[pl.ds(r * _L, _L)],
                         seg_sem)
    for r in range(_BPW):
        pltpu.make_async_copy(s_hbm.at[b0 + r],
                              psidx_v.at[pl.ds(r * _L, _L)], seg_sem).wait()

    iota = lax.iota(jnp.int32, 16)

    def idx_body(g, carry):
        f = g * 16
        s16 = psidx_v[pl.ds(f, 16)]
        psidx_v[pl.ds(f, 16)] = s16 * _L + lax.rem(f + iota, _L)
        return carry

    lax.fori_loop(0, _TPW // 16, idx_body, 0)

    def chunk_refs(r, sl):
        a_tok = tt_hbm.at[idx_v.at[r, pl.ds(0, _CHA)]]
        b_tok = tt_hbm.at[idx_v.at[r, pl.ds(_CHA, _CHB)]]
        a_ps = ps_hbm.at[psidx_v.at[pl.ds(r * _L, _CHA)]]
        b_ps = ps_hbm.at[psidx_v.at[pl.ds(r * _L + _CHA, _CHB)]]
        return a_tok, b_tok, a_ps, b_ps

    def out_refs(b):
        a_out = out_hbm.at[b, pl.ds(0, _CHA), pl.ds(0, _EMB)]
        b_out = out_hbm.at[b, pl.ds(_CHA, _CHB), pl.ds(0, _EMB)]
        return a_out, b_out

    def row_body(r, carry):
        b = b0 + r
        sl = lax.rem(r, 2)
        a_tok, b_tok, a_ps, b_ps = chunk_refs(r, sl)
        a_out, b_out = out_refs(b)

        @pl.when(r >= 2)
        def _():
            a_out2, b_out2 = out_refs(b - 2)
            pltpu.make_async_copy(bufa_v.at[sl], a_out2, wra_sems.at[sl]).wait()
            pltpu.make_async_copy(bufb_v.at[sl], b_out2, wrb_sems.at[sl]).wait()

        pltpu.async_copy(a_tok, bufa_v.at[sl], toka_sems.at[sl])
        pltpu.async_copy(b_tok, bufb_v.at[sl], tokb_sems.at[sl])
        pltpu.make_async_copy(a_tok, bufa_v.at[sl], toka_sems.at[sl]).wait()
        pltpu.async_copy(a_ps, bufa_v.at[sl], adda_sems.at[sl], add=True)
        pltpu.make_async_copy(b_tok, bufb_v.at[sl], tokb_sems.at[sl]).wait()
        pltpu.async_copy(b_ps, bufb_v.at[sl], addb_sems.at[sl], add=True)
        pltpu.make_async_copy(a_ps, bufa_v.at[sl], adda_sems.at[sl]).wait()
        pltpu.async_copy(bufa_v.at[sl], a_out, wra_sems.at[sl])
        pltpu.make_async_copy(b_ps, bufb_v.at[sl], addb_sems.at[sl]).wait()
        pltpu.async_copy(bufb_v.at[sl], b_out, wrb_sems.at[sl])
        return carry

    lax.fori_loop(0, _BPW, row_body, 0)

    for r in range(_BPW - 2, _BPW):
        sl = r % 2
        a_out, b_out = out_refs(b0 + r)
        pltpu.make_async_copy(bufa_v.at[sl], a_out, wra_sems.at[sl]).wait()
        pltpu.make_async_copy(bufb_v.at[sl], b_out, wrb_sems.at[sl]).wait()


def _sc_call(x, segment_label, token_table, posseg):
    mesh = plsc.VectorSubcoreMesh(core_axis_name="c", subcore_axis_name="s")
    fn = pl.kernel(
        _sc_body,
        out_type=jax.ShapeDtypeStruct((_B, _L, 2 * _EMB), jnp.float32),
        mesh=mesh,
        compiler_params=pltpu.CompilerParams(use_tc_tiling_on_sc=False),
        scratch_types=[
            pltpu.VMEM((_BPW, _L), jnp.int32),
            pltpu.VMEM((_TPW,), jnp.int32),
            pltpu.VMEM((2, _CHA, _EMB), jnp.float32),
            pltpu.VMEM((2, _CHB, _EMB), jnp.float32),
            pltpu.SemaphoreType.DMA((2,)),
            pltpu.SemaphoreType.DMA((2,)),
            pltpu.SemaphoreType.DMA((2,)),
            pltpu.SemaphoreType.DMA((2,)),
            pltpu.SemaphoreType.DMA((2,)),
            pltpu.SemaphoreType.DMA((2,)),
            pltpu.SemaphoreType.DMA,
        ],
    )
    return fn(x, segment_label, token_table, posseg)


def kernel(x, segment_label, token_table, pos_table, seg_table):
    posseg = _posseg(pos_table, seg_table)
    out = _sc_call(x.astype(jnp.int32), segment_label.astype(jnp.int32),
                    token_table, posseg)
    return out[:, :, :_EMB]
